# Initial kernel scaffold; baseline (speedup 1.0000x reference)
import jax
import jax.numpy as jnp
from jax.experimental import pallas as pl


def kernel(x, params):
    raise NotImplementedError("write your pallas kernel here")


# speedup vs baseline: 1.8042x; 1.8042x over previous
import jax
import jax.numpy as jnp
from jax.experimental import pallas as pl

_NC = 1024
_CD = 128
_IN = 104
_BM = 1024


def _ln(h, g, b):
    mu = jnp.mean(h, axis=-1, keepdims=True)
    var = jnp.mean((h - mu) ** 2, axis=-1, keepdims=True)
    return (h - mu) / jnp.sqrt(var + 1e-5) * g + b


def _vqvae_kernel(x_ref,
                  w1, b1, g1, be1,
                  w2, b2, g2, be2,
                  w3, b3, g3, be3,
                  w4, b4, g4, be4,
                  w5, b5, g5, be5,
                  w6, b6,
                  cb_ref, cbt_ref,
                  out_ref, idx_ref, commit_ref, ortho_ref):
    i = pl.program_id(0)

    @pl.when(i == 0)
    def _init():
        commit_ref[...] = jnp.zeros_like(commit_ref)
        cb = cb_ref[...]
        inv_n = 1.0 / (jnp.sqrt(jnp.sum(cb * cb, axis=1, keepdims=True)) + 1e-12)
        normed = cb * inv_n
        cbt = cbt_ref[...]
        inv_nr = 1.0 / (jnp.sqrt(jnp.sum(cbt * cbt, axis=0, keepdims=True)) + 1e-12)
        normed_t = cbt * inv_nr
        gram = jnp.dot(normed_t, normed, preferred_element_type=jnp.float32)
        ortho_ref[...] = jnp.sum(gram * gram).reshape(1, 1)

    h = x_ref[...]
    h = jax.nn.relu(_ln(jnp.dot(h, w1[...], preferred_element_type=jnp.float32) + b1[...], g1[...], be1[...]))
    h = jax.nn.relu(_ln(jnp.dot(h, w2[...], preferred_element_type=jnp.float32) + b2[...], g2[...], be2[...]))
    z = _ln(jnp.dot(h, w3[...], preferred_element_type=jnp.float32) + b3[...], g3[...], be3[...])

    cbt = cbt_ref[...]
    cross = jnp.dot(z, cbt, preferred_element_type=jnp.float32)
    csq = jnp.sum(cbt * cbt, axis=0, keepdims=True)
    zsq = jnp.sum(z * z, axis=1, keepdims=True)
    d = zsq - 2.0 * cross + csq
    dmin = jnp.min(d, axis=1, keepdims=True)
    iota = jax.lax.broadcasted_iota(jnp.int32, d.shape, 1)
    idxv = jnp.min(jnp.where(d == dmin, iota, _NC), axis=1)
    idx_ref[...] = idxv.reshape(idx_ref.shape)

    commit_ref[...] += jnp.sum(dmin).reshape(1, 1)

    onehot = (iota == idxv[:, None]).astype(jnp.float32)
    q = jnp.dot(onehot, cb_ref[...], preferred_element_type=jnp.float32)

    h = jax.nn.gelu(_ln(jnp.dot(q, w4[...], preferred_element_type=jnp.float32) + b4[...], g4[...], be4[...]))
    h = jax.nn.gelu(_ln(jnp.dot(h, w5[...], preferred_element_type=jnp.float32) + b5[...], g5[...], be5[...]))
    out_ref[...] = jnp.dot(h, w6[...], preferred_element_type=jnp.float32) + b6[...]


def kernel(x, params):
    batch = x.shape[0]
    xf = x.reshape(batch, _IN)
    enc, dec = params["enc"], params["dec"]
    cb = params["codebook"]
    cbt = cb.T

    nb = batch // _BM
    row = lambda v: v.reshape(1, -1)
    args = [xf]
    for layer in enc:
        args += [layer["W"], row(layer["b"]), row(layer["g"]), row(layer["beta"])]
    for layer in dec[:-1]:
        args += [layer["W"], row(layer["b"]), row(layer["g"]), row(layer["beta"])]
    args += [dec[-1]["W"], row(dec[-1]["b"]), cb, cbt]

    const = lambda a: pl.BlockSpec(a.shape, lambda i: (0,) * a.ndim)
    in_specs = [pl.BlockSpec((_BM, _IN), lambda i: (i, 0))]
    in_specs += [const(a) for a in args[1:]]

    out_shape = [
        jax.ShapeDtypeStruct((batch, _IN), jnp.float32),
        jax.ShapeDtypeStruct((nb, 1, _BM), jnp.int32),
        jax.ShapeDtypeStruct((1, 1), jnp.float32),
        jax.ShapeDtypeStruct((1, 1), jnp.float32),
    ]
    out_specs = [
        pl.BlockSpec((_BM, _IN), lambda i: (i, 0)),
        pl.BlockSpec((1, 1, _BM), lambda i: (i, 0, 0)),
        pl.BlockSpec((1, 1), lambda i: (0, 0)),
        pl.BlockSpec((1, 1), lambda i: (0, 0)),
    ]

    out, idx, commit, ortho = pl.pallas_call(
        _vqvae_kernel,
        grid=(nb,),
        in_specs=in_specs,
        out_specs=out_specs,
        out_shape=out_shape,
    )(*args)

    x_recon = out.reshape(batch, 52, 2)
    indices = idx.reshape(batch)
    commit_loss = commit[0, 0] / (batch * _CD)
    ortho_loss = ortho[0, 0] / (_NC * _NC) - 1.0 / _NC
    vq_loss = 0.1 * commit_loss + 1.0 * ortho_loss
    return (x_recon, vq_loss, indices)


# speedup vs baseline: 1.9411x; 1.0759x over previous
import jax
import jax.numpy as jnp
from jax.experimental import pallas as pl

_NC = 1024
_CD = 128
_IN = 104
_BM = 2048


def _ln(h, g, b):
    mu = jnp.mean(h, axis=-1, keepdims=True)
    var = jnp.mean((h - mu) ** 2, axis=-1, keepdims=True)
    return (h - mu) / jnp.sqrt(var + 1e-5) * g + b


def _vqvae_kernel(x_ref,
                  w1, b1, g1, be1,
                  w2, b2, g2, be2,
                  w3, b3, g3, be3,
                  w4, b4, g4, be4,
                  w5, b5, g5, be5,
                  w6, b6,
                  cb_ref, cbt_ref,
                  out_ref, idx_ref, commit_ref, ortho_ref):
    i = pl.program_id(0)

    @pl.when(i == 0)
    def _init():
        commit_ref[...] = jnp.zeros_like(commit_ref)
        cb = cb_ref[...]
        inv_n = 1.0 / (jnp.sqrt(jnp.sum(cb * cb, axis=1, keepdims=True)) + 1e-12)
        normed = cb * inv_n
        cbt = cbt_ref[...]
        inv_nr = 1.0 / (jnp.sqrt(jnp.sum(cbt * cbt, axis=0, keepdims=True)) + 1e-12)
        normed_t = cbt * inv_nr
        gram = jnp.dot(normed_t, normed, preferred_element_type=jnp.float32)
        ortho_ref[...] = jnp.sum(gram * gram).reshape(1, 1)

    h = x_ref[...]
    h = jax.nn.relu(_ln(jnp.dot(h, w1[...], preferred_element_type=jnp.float32) + b1[...], g1[...], be1[...]))
    h = jax.nn.relu(_ln(jnp.dot(h, w2[...], preferred_element_type=jnp.float32) + b2[...], g2[...], be2[...]))
    z = _ln(jnp.dot(h, w3[...], preferred_element_type=jnp.float32) + b3[...], g3[...], be3[...])

    cbt = cbt_ref[...]
    cross = jnp.dot(z, cbt, preferred_element_type=jnp.float32)
    c2 = 0.5 * jnp.sum(cbt * cbt, axis=0, keepdims=True)
    s = cross - c2
    smax = jnp.max(s, axis=1, keepdims=True)
    iota = jax.lax.broadcasted_iota(jnp.int32, s.shape, 1)
    idxv = jnp.min(jnp.where(s == smax, iota, _NC), axis=1)
    idx_ref[...] = idxv.reshape(idx_ref.shape)

    commit_ref[...] += (jnp.sum(z * z) - 2.0 * jnp.sum(smax)).reshape(1, 1)

    onehot = (iota == idxv[:, None]).astype(jnp.float32)
    q = jnp.dot(onehot, cb_ref[...], preferred_element_type=jnp.float32)

    h = jax.nn.gelu(_ln(jnp.dot(q, w4[...], preferred_element_type=jnp.float32) + b4[...], g4[...], be4[...]))
    h = jax.nn.gelu(_ln(jnp.dot(h, w5[...], preferred_element_type=jnp.float32) + b5[...], g5[...], be5[...]))
    out_ref[...] = jnp.dot(h, w6[...], preferred_element_type=jnp.float32) + b6[...]


def kernel(x, params):
    batch = x.shape[0]
    xf = x.reshape(batch, _IN)
    enc, dec = params["enc"], params["dec"]
    cb = params["codebook"]
    cbt = cb.T

    nb = batch // _BM
    row = lambda v: v.reshape(1, -1)
    args = [xf]
    for layer in enc:
        args += [layer["W"], row(layer["b"]), row(layer["g"]), row(layer["beta"])]
    for layer in dec[:-1]:
        args += [layer["W"], row(layer["b"]), row(layer["g"]), row(layer["beta"])]
    args += [dec[-1]["W"], row(dec[-1]["b"]), cb, cbt]

    const = lambda a: pl.BlockSpec(a.shape, lambda i: (0,) * a.ndim)
    in_specs = [pl.BlockSpec((_BM, _IN), lambda i: (i, 0))]
    in_specs += [const(a) for a in args[1:]]

    out_shape = [
        jax.ShapeDtypeStruct((batch, _IN), jnp.float32),
        jax.ShapeDtypeStruct((nb, 1, _BM), jnp.int32),
        jax.ShapeDtypeStruct((1, 1), jnp.float32),
        jax.ShapeDtypeStruct((1, 1), jnp.float32),
    ]
    out_specs = [
        pl.BlockSpec((_BM, _IN), lambda i: (i, 0)),
        pl.BlockSpec((1, 1, _BM), lambda i: (i, 0, 0)),
        pl.BlockSpec((1, 1), lambda i: (0, 0)),
        pl.BlockSpec((1, 1), lambda i: (0, 0)),
    ]

    out, idx, commit, ortho = pl.pallas_call(
        _vqvae_kernel,
        grid=(nb,),
        in_specs=in_specs,
        out_specs=out_specs,
        out_shape=out_shape,
    )(*args)

    x_recon = out.reshape(batch, 52, 2)
    indices = idx.reshape(batch)
    commit_loss = commit[0, 0] / (batch * _CD)
    ortho_loss = ortho[0, 0] / (_NC * _NC) - 1.0 / _NC
    vq_loss = 0.1 * commit_loss + 1.0 * ortho_loss
    return (x_recon, vq_loss, indices)
